# Initial kernel scaffold; baseline (speedup 1.0000x reference)
import jax
import jax.numpy as jnp
from jax.experimental import pallas as pl


def kernel(x, edge_index, edge_attn, W1, b1, W2, b2):
    raise NotImplementedError("write your pallas kernel here")


# speedup vs baseline: 3.0046x; 3.0046x over previous
import functools

import jax
import jax.numpy as jnp
from jax import lax
from jax.experimental import pallas as pl
from jax.experimental.pallas import tpu as pltpu
from jax.experimental.pallas import tpu_sc as plsc

N = 10000
E = 320000
D = 128

NC = 2
NS = 16
CHUNK = 128
N_TILES = NC * NS
CPT = (-(-E // (N_TILES * CHUNK)) + 7) // 8 * 8
EPT = CPT * CHUNK
E_PAD = N_TILES * EPT
NP = 10240
RPT = NP // NS

_sc_mesh = plsc.VectorSubcoreMesh(core_axis_name="c", subcore_axis_name="s")


@functools.partial(
    pl.kernel,
    out_type=jax.ShapeDtypeStruct((NC, NP, D), jnp.float32),
    mesh=_sc_mesh,
    scratch_types=[
        pltpu.VMEM_SHARED((NP, D), jnp.float32),
        pltpu.VMEM((CPT, CHUNK), jnp.int32),
        pltpu.VMEM((CPT, CHUNK), jnp.int32),
        pltpu.VMEM((EPT,), jnp.float32),
        pltpu.VMEM((CHUNK, D), jnp.float32),
        pltpu.SemaphoreType.DMA,
    ],
)
def _sc_message(x_hbm, src_hbm, dst_hbm, attn_hbm, out_hbm,
                hn_sh, src_v, dst_v, attn_v, rows_v, gsem):
    c = lax.axis_index("c")
    s = lax.axis_index("s")
    base_chunk = (c * NS + s) * CPT

    pltpu.sync_copy(src_hbm.at[pl.ds(base_chunk, CPT), :], src_v)
    pltpu.sync_copy(dst_hbm.at[pl.ds(base_chunk, CPT), :], dst_v)
    pltpu.sync_copy(attn_hbm.at[pl.ds(base_chunk * CHUNK, EPT)], attn_v)

    zero16 = jnp.zeros((16,), jnp.float32)

    def zstore(i, carry):
        rows_v[i // 8, pl.ds((i % 8) * 16, 16)] = zero16
        return carry

    lax.fori_loop(0, CHUNK * 8, zstore, 0)
    for j in range(RPT // CHUNK):
        pltpu.sync_copy(rows_v, hn_sh.at[pl.ds(s * RPT + j * CHUNK, CHUNK), :])
    plsc.subcore_barrier()

    def chunk_body(k, carry):
        pltpu.async_copy(x_hbm.at[src_v.at[k]], rows_v, gsem).wait()

        def group_body(g, carry2):
            av = attn_v[pl.ds(k * CHUNK + g * 16, 16)]
            for j in range(16):
                e = g * 16 + j
                a = jnp.broadcast_to(av[j], (16,))
                for db in range(D // 16):
                    sl = pl.ds(db * 16, 16)
                    rows_v[e, sl] = rows_v[e, sl] * a
            return carry2

        lax.fori_loop(0, CHUNK // 16, group_body, 0)
        pltpu.sync_copy(rows_v, hn_sh.at[dst_v.at[k]], add=True)
        return carry

    lax.fori_loop(0, CPT, chunk_body, 0)
    plsc.subcore_barrier()

    pltpu.sync_copy(hn_sh.at[pl.ds(s * RPT, RPT), :],
                    out_hbm.at[c, pl.ds(s * RPT, RPT), :])


def _tc_body(x_ref, p_ref, w1_ref, b1_ref, w2_ref, b2_ref, o_ref):
    x = x_ref[...]
    h = p_ref[0] + p_ref[1]
    dn = (((1,), (1,)), ((), ()))
    a = lax.dot_general(x + h, w1_ref[...], dn,
                        preferred_element_type=jnp.float32) + b1_ref[...]
    b = lax.dot_general(x * h, w2_ref[...], dn,
                        preferred_element_type=jnp.float32) + b2_ref[...]
    o_ref[...] = jnp.where(a > 0, a, 0.01 * a) + jnp.where(b > 0, b, 0.01 * b)


_TC_BLK = 2000


def _bi_interaction(x, partials, W1, b1, W2, b2):
    return pl.pallas_call(
        _tc_body,
        grid=(N // _TC_BLK,),
        in_specs=[
            pl.BlockSpec((_TC_BLK, D), lambda i: (i, 0)),
            pl.BlockSpec((NC, _TC_BLK, D), lambda i: (0, i, 0)),
            pl.BlockSpec((D, D), lambda i: (0, 0)),
            pl.BlockSpec((1, D), lambda i: (0, 0)),
            pl.BlockSpec((D, D), lambda i: (0, 0)),
            pl.BlockSpec((1, D), lambda i: (0, 0)),
        ],
        out_specs=pl.BlockSpec((_TC_BLK, D), lambda i: (i, 0)),
        out_shape=jax.ShapeDtypeStruct((N, D), jnp.float32),
    )(x, partials, W1, b1.reshape(1, D), W2, b2.reshape(1, D))


def kernel(x, edge_index, edge_attn, W1, b1, W2, b2):
    src = edge_index[0]
    dst = edge_index[1]
    attn = edge_attn[:, 0]
    pad = E_PAD - E
    src_p = jnp.concatenate(
        [src, jnp.zeros((pad,), jnp.int32)]).reshape(E_PAD // CHUNK, CHUNK)
    dst_p = jnp.concatenate(
        [dst, jnp.zeros((pad,), jnp.int32)]).reshape(E_PAD // CHUNK, CHUNK)
    attn_p = jnp.concatenate([attn, jnp.zeros((pad,), jnp.float32)])
    partials = _sc_message(x, src_p, dst_p, attn_p)
    return _bi_interaction(x, partials, W1, b1, W2, b2)


# speedup vs baseline: 3.0666x; 1.0206x over previous
import functools

import jax
import jax.numpy as jnp
from jax import lax
from jax.experimental import pallas as pl
from jax.experimental.pallas import tpu as pltpu
from jax.experimental.pallas import tpu_sc as plsc

N = 10000
E = 320000
D = 128

NC = 2
NS = 16
CHUNK = 64
N_TILES = NC * NS
CPT = (-(-E // (N_TILES * CHUNK)) + 7) // 8 * 8
EPT = CPT * CHUNK
E_PAD = N_TILES * EPT
NP = 10240
RPT = NP // NS

_sc_mesh = plsc.VectorSubcoreMesh(core_axis_name="c", subcore_axis_name="s")


@functools.partial(
    pl.kernel,
    out_type=jax.ShapeDtypeStruct((NC, NP, D), jnp.float32),
    mesh=_sc_mesh,
    scratch_types=[
        pltpu.VMEM_SHARED((NP, D), jnp.float32),
        pltpu.VMEM((CPT, CHUNK), jnp.int32),
        pltpu.VMEM((EPT,), jnp.float32),
        pltpu.VMEM((CHUNK, D), jnp.float32),
        pltpu.VMEM((CHUNK, D), jnp.float32),
        pltpu.VMEM((CHUNK,), jnp.int32),
        pltpu.VMEM((CHUNK,), jnp.int32),
        pltpu.VMEM((CHUNK,), jnp.int32),
        pltpu.VMEM((CHUNK,), jnp.int32),
        pltpu.SemaphoreType.DMA,
        pltpu.SemaphoreType.DMA,
        pltpu.SemaphoreType.DMA,
        pltpu.SemaphoreType.DMA,
    ],
)
def _sc_message(x_hbm, packed_hbm, attn_hbm, out_hbm,
                hn_sh, packed_v, attn_v, rows0_v, rows1_v,
                si0, si1, di0, di1, g0, g1, s0, s1):
    c = lax.axis_index("c")
    s = lax.axis_index("s")
    base_chunk = (c * NS + s) * CPT

    pltpu.sync_copy(packed_hbm.at[pl.ds(base_chunk, CPT), :], packed_v)
    pltpu.sync_copy(attn_hbm.at[pl.ds(base_chunk * CHUNK, EPT)], attn_v)

    zero16 = jnp.zeros((16,), jnp.float32)

    def zstore(i, carry):
        rows0_v[i // 8, pl.ds((i % 8) * 16, 16)] = zero16
        return carry

    lax.fori_loop(0, CHUNK * 8, zstore, 0)
    for j in range(RPT // CHUNK):
        pltpu.sync_copy(rows0_v, hn_sh.at[pl.ds(s * RPT + j * CHUNK, CHUNK), :])
    plsc.subcore_barrier()

    def scale(rows, k):
        def group_body(g, carry2):
            av = attn_v[pl.ds(k * CHUNK + g * 16, 16)]
            for j in range(16):
                e = g * 16 + j
                a = jnp.broadcast_to(av[j], (16,))
                for db in range(D // 16):
                    sl = pl.ds(db * 16, 16)
                    rows[e, sl] = rows[e, sl] * a
            return carry2

        lax.fori_loop(0, CHUNK // 16, group_body, 0)

    def unpack(k, sidx, didx):
        for g in range(CHUNK // 16):
            sl = pl.ds(g * 16, 16)
            v = packed_v[k, sl]
            sidx[sl] = lax.shift_right_logical(v, 14)
            didx[sl] = lax.bitwise_and(v, jnp.int32(16383))

    def sg(rows, sidx, sem):
        pltpu.async_copy(x_hbm.at[sidx], rows, sem)

    def wg(rows, sidx, sem):
        pltpu.make_async_copy(x_hbm.at[sidx], rows, sem).wait()

    def ss(rows, didx, sem):
        pltpu.async_copy(rows, hn_sh.at[didx], sem, add=True)

    def ws(rows, didx, sem):
        pltpu.make_async_copy(rows, hn_sh.at[didx], sem).wait()

    unpack(0, si0, di0)
    sg(rows0_v, si0, g0)
    wg(rows0_v, si0, g0)
    scale(rows0_v, 0)
    unpack(1, si1, di1)
    sg(rows1_v, si1, g1)
    ss(rows0_v, di0, s0)

    def pair_body(kk, carry):
        a = 2 * kk + 1
        b = 2 * kk + 2
        wg(rows1_v, si1, g1)
        scale(rows1_v, a)
        ws(rows0_v, di0, s0)
        unpack(a + 1, si0, di0)
        sg(rows0_v, si0, g0)
        ss(rows1_v, di1, s1)
        wg(rows0_v, si0, g0)
        scale(rows0_v, b)
        ws(rows1_v, di1, s1)
        unpack(b + 1, si1, di1)
        sg(rows1_v, si1, g1)
        ss(rows0_v, di0, s0)
        return carry

    lax.fori_loop(0, (CPT - 2) // 2, pair_body, 0)

    last = CPT - 1
    wg(rows1_v, si1, g1)
    scale(rows1_v, last)
    ws(rows0_v, di0, s0)
    ss(rows1_v, di1, s1)
    ws(rows1_v, di1, s1)
    plsc.subcore_barrier()

    pltpu.sync_copy(hn_sh.at[pl.ds(s * RPT, RPT), :],
                    out_hbm.at[c, pl.ds(s * RPT, RPT), :])


def _tc_body(x_ref, p_ref, w1_ref, b1_ref, w2_ref, b2_ref, o_ref):
    x = x_ref[...]
    h = p_ref[0] + p_ref[1]
    dn = (((1,), (1,)), ((), ()))
    a = lax.dot_general(x + h, w1_ref[...], dn,
                        preferred_element_type=jnp.float32) + b1_ref[...]
    b = lax.dot_general(x * h, w2_ref[...], dn,
                        preferred_element_type=jnp.float32) + b2_ref[...]
    o_ref[...] = jnp.where(a > 0, a, 0.01 * a) + jnp.where(b > 0, b, 0.01 * b)


_TC_BLK = 2000


def _bi_interaction(x, partials, W1, b1, W2, b2):
    return pl.pallas_call(
        _tc_body,
        grid=(N // _TC_BLK,),
        in_specs=[
            pl.BlockSpec((_TC_BLK, D), lambda i: (i, 0)),
            pl.BlockSpec((NC, _TC_BLK, D), lambda i: (0, i, 0)),
            pl.BlockSpec((D, D), lambda i: (0, 0)),
            pl.BlockSpec((1, D), lambda i: (0, 0)),
            pl.BlockSpec((D, D), lambda i: (0, 0)),
            pl.BlockSpec((1, D), lambda i: (0, 0)),
        ],
        out_specs=pl.BlockSpec((_TC_BLK, D), lambda i: (i, 0)),
        out_shape=jax.ShapeDtypeStruct((N, D), jnp.float32),
    )(x, partials, W1, b1.reshape(1, D), W2, b2.reshape(1, D))


def kernel(x, edge_index, edge_attn, W1, b1, W2, b2):
    src = edge_index[0]
    dst = edge_index[1]
    attn = edge_attn[:, 0]
    pad = E_PAD - E
    packed = src * jnp.int32(16384) + dst
    packed_p = jnp.concatenate(
        [packed, jnp.zeros((pad,), jnp.int32)]).reshape(E_PAD // CHUNK, CHUNK)
    attn_p = jnp.concatenate([attn, jnp.zeros((pad,), jnp.float32)])
    partials = _sc_message(x, packed_p, attn_p)
    return _bi_interaction(x, partials, W1, b1, W2, b2)


# speedup vs baseline: 3.4065x; 1.1108x over previous
import functools

import jax
import jax.numpy as jnp
from jax import lax
from jax.experimental import pallas as pl
from jax.experimental.pallas import tpu as pltpu
from jax.experimental.pallas import tpu_sc as plsc

N = 10000
E = 320000
D = 128

NC = 2
NS = 16
CHUNK = 64
N_TILES = NC * NS
CPT0 = 208
CPT1 = 112
CPT_MAX = CPT0
EPT0 = CPT0 * CHUNK
EPT1 = CPT1 * CHUNK
E_PAD = NS * (EPT0 + EPT1)
E_ALLOC = E_PAD + (CPT0 - CPT1) * CHUNK
NP = 10240
RPT = NP // NS

_sc_mesh = plsc.VectorSubcoreMesh(core_axis_name="c", subcore_axis_name="s")


@functools.partial(
    pl.kernel,
    out_type=jax.ShapeDtypeStruct((NC, NP, D), jnp.float32),
    mesh=_sc_mesh,
    scratch_types=[
        pltpu.VMEM_SHARED((NP, D), jnp.float32),
        pltpu.VMEM((CPT_MAX // 2, 128), jnp.int32),
        pltpu.VMEM((EPT0,), jnp.float32),
        pltpu.VMEM((CHUNK, D), jnp.float32),
        pltpu.VMEM((CHUNK, D), jnp.float32),
        pltpu.VMEM((CHUNK,), jnp.int32),
        pltpu.VMEM((CHUNK,), jnp.int32),
        pltpu.VMEM((CHUNK,), jnp.int32),
        pltpu.VMEM((CHUNK,), jnp.int32),
        pltpu.SemaphoreType.DMA,
        pltpu.SemaphoreType.DMA,
        pltpu.SemaphoreType.DMA,
        pltpu.SemaphoreType.DMA,
    ],
)
def _sc_message(x_hbm, packed_hbm, attn_hbm, out_hbm,
                hn_sh, packed_v, attn_v, rows0_v, rows1_v,
                si0, si1, di0, di1, g0, g1, s0, s1):
    c = lax.axis_index("c")
    s = lax.axis_index("s")
    base_chunk = c * (NS * CPT0) + s * CPT1 + (1 - c) * s * (CPT0 - CPT1)
    base_chunk = pl.multiple_of(base_chunk, 8)

    brow = pl.multiple_of(base_chunk // 2, 8)
    pltpu.sync_copy(packed_hbm.at[pl.ds(brow, CPT_MAX // 2), :], packed_v)
    pltpu.sync_copy(attn_hbm.at[pl.ds(base_chunk * CHUNK, EPT0)], attn_v)

    zero16 = jnp.zeros((16,), jnp.float32)

    def zstore(i, carry):
        rows0_v[i // 8, pl.ds((i % 8) * 16, 16)] = zero16
        return carry

    lax.fori_loop(0, CHUNK * 8, zstore, 0)
    for j in range(RPT // CHUNK):
        pltpu.sync_copy(rows0_v, hn_sh.at[pl.ds(s * RPT + j * CHUNK, CHUNK), :])
    plsc.subcore_barrier()

    def scale(rows, k):
        def group_body(g, carry2):
            av = attn_v[pl.ds(k * CHUNK + g * 16, 16)]
            for j in range(16):
                e = g * 16 + j
                a = jnp.broadcast_to(av[j], (16,))
                for db in range(D // 16):
                    sl = pl.ds(db * 16, 16)
                    rows[e, sl] = rows[e, sl] * a
            return carry2

        lax.fori_loop(0, CHUNK // 16, group_body, 0)

    def unpack(k, sidx, didx):
        for g in range(CHUNK // 16):
            sl = pl.ds(g * 16, 16)
            v = packed_v[k // 2, pl.ds((k % 2) * 64 + g * 16, 16)]
            sidx[sl] = lax.shift_right_logical(v, 14)
            didx[sl] = lax.bitwise_and(v, jnp.int32(16383))

    def sg(rows, sidx, sem):
        pltpu.async_copy(x_hbm.at[sidx], rows, sem)

    def wg(rows, sidx, sem):
        pltpu.make_async_copy(x_hbm.at[sidx], rows, sem).wait()

    def ss(rows, didx, sem):
        pltpu.async_copy(rows, hn_sh.at[didx], sem, add=True)

    def ws(rows, didx, sem):
        pltpu.make_async_copy(rows, hn_sh.at[didx], sem).wait()

    unpack(0, si0, di0)
    sg(rows0_v, si0, g0)
    wg(rows0_v, si0, g0)
    scale(rows0_v, 0)
    unpack(1, si1, di1)
    sg(rows1_v, si1, g1)
    ss(rows0_v, di0, s0)

    def pair_body(kk, carry):
        a = 2 * kk + 1
        b = 2 * kk + 2
        wg(rows1_v, si1, g1)
        scale(rows1_v, a)
        ws(rows0_v, di0, s0)
        unpack(a + 1, si0, di0)
        sg(rows0_v, si0, g0)
        ss(rows1_v, di1, s1)
        wg(rows0_v, si0, g0)
        scale(rows0_v, b)
        ws(rows1_v, di1, s1)
        unpack(b + 1, si1, di1)
        sg(rows1_v, si1, g1)
        ss(rows0_v, di0, s0)
        return carry

    lax.fori_loop(0, (CPT1 - 2) // 2, pair_body, 0)

    @pl.when(c == 0)
    def _():
        lax.fori_loop((CPT1 - 2) // 2,
                      (CPT1 - 2) // 2 + (CPT0 - CPT1) // 2, pair_body, 0)

    last = (CPT1 - 1) + (1 - c) * (CPT0 - CPT1)
    wg(rows1_v, si1, g1)
    scale(rows1_v, last)
    ws(rows0_v, di0, s0)
    ss(rows1_v, di1, s1)
    ws(rows1_v, di1, s1)
    plsc.subcore_barrier()

    pltpu.sync_copy(hn_sh.at[pl.ds(s * RPT, RPT), :],
                    out_hbm.at[c, pl.ds(s * RPT, RPT), :])


def _tc_body(x_ref, p_ref, w1_ref, b1_ref, w2_ref, b2_ref, o_ref):
    x = x_ref[...]
    h = p_ref[0] + p_ref[1]
    dn = (((1,), (1,)), ((), ()))
    a = lax.dot_general(x + h, w1_ref[...], dn,
                        preferred_element_type=jnp.float32) + b1_ref[...]
    b = lax.dot_general(x * h, w2_ref[...], dn,
                        preferred_element_type=jnp.float32) + b2_ref[...]
    o_ref[...] = jnp.where(a > 0, a, 0.01 * a) + jnp.where(b > 0, b, 0.01 * b)


_TC_BLK = 2000


def _bi_interaction(x, partials, W1, b1, W2, b2):
    return pl.pallas_call(
        _tc_body,
        grid=(N // _TC_BLK,),
        in_specs=[
            pl.BlockSpec((_TC_BLK, D), lambda i: (i, 0)),
            pl.BlockSpec((NC, _TC_BLK, D), lambda i: (0, i, 0)),
            pl.BlockSpec((D, D), lambda i: (0, 0)),
            pl.BlockSpec((1, D), lambda i: (0, 0)),
            pl.BlockSpec((D, D), lambda i: (0, 0)),
            pl.BlockSpec((1, D), lambda i: (0, 0)),
        ],
        out_specs=pl.BlockSpec((_TC_BLK, D), lambda i: (i, 0)),
        out_shape=jax.ShapeDtypeStruct((N, D), jnp.float32),
    )(x, partials, W1, b1.reshape(1, D), W2, b2.reshape(1, D))


def kernel(x, edge_index, edge_attn, W1, b1, W2, b2):
    src = edge_index[0]
    dst = edge_index[1]
    attn = edge_attn[:, 0]
    pad = E_ALLOC - E
    packed = src * jnp.int32(16384) + dst
    packed_p = jnp.concatenate(
        [packed, jnp.zeros((pad,), jnp.int32)]).reshape(E_ALLOC // 128, 128)
    attn_p = jnp.concatenate([attn, jnp.zeros((pad,), jnp.float32)])
    partials = _sc_message(x, packed_p, attn_p)
    return _bi_interaction(x, partials, W1, b1, W2, b2)


# speedup vs baseline: 3.8281x; 1.1238x over previous
import functools

import jax
import jax.numpy as jnp
from jax import lax
from jax.experimental import pallas as pl
from jax.experimental.pallas import tpu as pltpu
from jax.experimental.pallas import tpu_sc as plsc

N = 10000
E = 320000
D = 128

NC = 2
NS = 16
CHUNK = 32
N_TILES = NC * NS
CPT0 = 416
CPT1 = 224
CPT_MAX = CPT0
NB = 4
EPT0 = CPT0 * CHUNK
EPT1 = CPT1 * CHUNK
E_PAD = NS * (EPT0 + EPT1)
E_ALLOC = E_PAD + (CPT0 - CPT1) * CHUNK
NP = 10240
RPT = NP // NS

_sc_mesh = plsc.VectorSubcoreMesh(core_axis_name="c", subcore_axis_name="s")


@functools.partial(
    pl.kernel,
    out_type=jax.ShapeDtypeStruct((NC, NP, D), jnp.float32),
    mesh=_sc_mesh,
    scratch_types=[
        pltpu.VMEM_SHARED((NP, D), jnp.float32),
        pltpu.VMEM((CPT_MAX // 4, 128), jnp.int32),
        pltpu.VMEM((EPT0,), jnp.float32),
        [pltpu.VMEM((CHUNK, D), jnp.float32) for _ in range(NB)],
        [pltpu.VMEM((CHUNK,), jnp.int32) for _ in range(NB)],
        [pltpu.VMEM((CHUNK,), jnp.int32) for _ in range(NB)],
        [pltpu.SemaphoreType.DMA for _ in range(NB)],
        [pltpu.SemaphoreType.DMA for _ in range(NB)],
    ],
)
def _sc_message(x_hbm, packed_hbm, attn_hbm, out_hbm,
                hn_sh, packed_v, attn_v, rows, si, di, gs, ss_sems):
    c = lax.axis_index("c")
    s = lax.axis_index("s")
    base_chunk = c * (NS * CPT0) + s * CPT1 + (1 - c) * s * (CPT0 - CPT1)
    base_chunk = pl.multiple_of(base_chunk, 32)

    brow = pl.multiple_of(base_chunk // 4, 8)
    pltpu.sync_copy(packed_hbm.at[pl.ds(brow, CPT_MAX // 4), :], packed_v)
    pltpu.sync_copy(attn_hbm.at[pl.ds(base_chunk * CHUNK, EPT0)], attn_v)

    zero16 = jnp.zeros((16,), jnp.float32)

    def zstore(i, carry):
        rows[0][i // 8, pl.ds((i % 8) * 16, 16)] = zero16
        return carry

    lax.fori_loop(0, CHUNK * 8, zstore, 0)
    for j in range(RPT // CHUNK):
        pltpu.sync_copy(rows[0], hn_sh.at[pl.ds(s * RPT + j * CHUNK, CHUNK), :])
    plsc.subcore_barrier()

    def scale(buf, k):
        def group_body(g, carry2):
            av = attn_v[pl.ds(k * CHUNK + g * 16, 16)]
            for j in range(16):
                e = g * 16 + j
                a = jnp.broadcast_to(av[j], (16,))
                for db in range(D // 16):
                    sl = pl.ds(db * 16, 16)
                    rows[buf][e, sl] = rows[buf][e, sl] * a
            return carry2

        lax.fori_loop(0, CHUNK // 16, group_body, 0)

    def unpack(k, buf):
        for g in range(CHUNK // 16):
            sl = pl.ds(g * 16, 16)
            v = packed_v[k // 4, pl.ds((k % 4) * 32 + g * 16, 16)]
            si[buf][sl] = lax.shift_right_logical(v, 14)
            di[buf][sl] = lax.bitwise_and(v, jnp.int32(16383))

    def sg(k, buf):
        pltpu.async_copy(x_hbm.at[si[buf]], rows[buf], gs[buf])

    def wg(buf):
        pltpu.make_async_copy(x_hbm.at[si[buf]], rows[buf], gs[buf]).wait()

    def ss(buf):
        pltpu.async_copy(rows[buf], hn_sh.at[di[buf]], ss_sems[buf], add=True)

    def ws(buf):
        pltpu.make_async_copy(rows[buf], hn_sh.at[di[buf]], ss_sems[buf]).wait()

    for k0 in range(NB - 1):
        unpack(k0, k0)
        sg(k0, k0)
    wg(0)
    scale(0, 0)
    unpack(NB - 1, NB - 1)
    sg(NB - 1, NB - 1)
    ss(0)

    def step(k, m):
        wg(m)
        scale(m, k)
        ws((m + NB - 1) % NB)
        unpack(k + NB - 1, (m + NB - 1) % NB)
        sg(k + NB - 1, (m + NB - 1) % NB)
        ss(m)

    def quad_body(kk, carry):
        for i in range(NB):
            k = NB * kk + 1 + i
            step(k, (1 + i) % NB)
        return carry

    lax.fori_loop(0, (CPT1 - NB) // NB, quad_body, 0)

    @pl.when(c == 0)
    def _():
        lax.fori_loop((CPT1 - NB) // NB, (CPT0 - NB) // NB, quad_body, 0)

    last = (CPT1 - 1) + (1 - c) * (CPT0 - CPT1)
    for i in range(NB - 1):
        k = last - (NB - 2) + i
        m = 1 + i
        wg(m)
        scale(m, k)
        ws(m - 1)
        ss(m)
    ws(NB - 1)
    plsc.subcore_barrier()

    pltpu.sync_copy(hn_sh.at[pl.ds(s * RPT, RPT), :],
                    out_hbm.at[c, pl.ds(s * RPT, RPT), :])


def _tc_body(x_ref, p_ref, w1_ref, b1_ref, w2_ref, b2_ref, o_ref):
    x = x_ref[...]
    h = p_ref[0] + p_ref[1]
    dn = (((1,), (1,)), ((), ()))
    a = lax.dot_general(x + h, w1_ref[...], dn,
                        preferred_element_type=jnp.float32) + b1_ref[...]
    b = lax.dot_general(x * h, w2_ref[...], dn,
                        preferred_element_type=jnp.float32) + b2_ref[...]
    o_ref[...] = jnp.where(a > 0, a, 0.01 * a) + jnp.where(b > 0, b, 0.01 * b)


_TC_BLK = 2000


def _bi_interaction(x, partials, W1, b1, W2, b2):
    return pl.pallas_call(
        _tc_body,
        grid=(N // _TC_BLK,),
        in_specs=[
            pl.BlockSpec((_TC_BLK, D), lambda i: (i, 0)),
            pl.BlockSpec((NC, _TC_BLK, D), lambda i: (0, i, 0)),
            pl.BlockSpec((D, D), lambda i: (0, 0)),
            pl.BlockSpec((1, D), lambda i: (0, 0)),
            pl.BlockSpec((D, D), lambda i: (0, 0)),
            pl.BlockSpec((1, D), lambda i: (0, 0)),
        ],
        out_specs=pl.BlockSpec((_TC_BLK, D), lambda i: (i, 0)),
        out_shape=jax.ShapeDtypeStruct((N, D), jnp.float32),
    )(x, partials, W1, b1.reshape(1, D), W2, b2.reshape(1, D))


def kernel(x, edge_index, edge_attn, W1, b1, W2, b2):
    src = edge_index[0]
    dst = edge_index[1]
    attn = edge_attn[:, 0]
    pad = E_ALLOC - E
    packed = src * jnp.int32(16384) + dst
    packed_p = jnp.concatenate(
        [packed, jnp.zeros((pad,), jnp.int32)]).reshape(E_ALLOC // 128, 128)
    attn_p = jnp.concatenate([attn, jnp.zeros((pad,), jnp.float32)])
    partials = _sc_message(x, packed_p, attn_p)
    return _bi_interaction(x, partials, W1, b1, W2, b2)


# speedup vs baseline: 3.9218x; 1.0245x over previous
import functools

import jax
import jax.numpy as jnp
from jax import lax
from jax.experimental import pallas as pl
from jax.experimental.pallas import tpu as pltpu
from jax.experimental.pallas import tpu_sc as plsc

N = 10000
E = 320000
D = 128

NC = 2
NS = 16
CHUNK = 32
N_TILES = NC * NS
CPT0 = 512
CPT1 = 128
CPT_MAX = CPT0
NB = 4
EPT0 = CPT0 * CHUNK
EPT1 = CPT1 * CHUNK
E_PAD = NS * (EPT0 + EPT1)
E_ALLOC = E_PAD + (CPT0 - CPT1) * CHUNK
NP = 10240
RPT = NP // NS

_sc_mesh = plsc.VectorSubcoreMesh(core_axis_name="c", subcore_axis_name="s")


@functools.partial(
    pl.kernel,
    out_type=jax.ShapeDtypeStruct((NC, NP, D), jnp.float32),
    mesh=_sc_mesh,
    scratch_types=[
        pltpu.VMEM_SHARED((NP, D), jnp.float32),
        pltpu.VMEM((CPT_MAX // 4, 128), jnp.int32),
        [pltpu.VMEM((CHUNK,), jnp.float32) for _ in range(NB)],
        [pltpu.VMEM((CHUNK, D), jnp.float32) for _ in range(NB)],
        [pltpu.VMEM((CHUNK,), jnp.int32) for _ in range(NB)],
        [pltpu.VMEM((CHUNK,), jnp.int32) for _ in range(NB)],
        [pltpu.SemaphoreType.DMA for _ in range(NB)],
        [pltpu.SemaphoreType.DMA for _ in range(NB)],
        [pltpu.SemaphoreType.DMA for _ in range(NB)],
    ],
)
def _sc_message(x_hbm, packed_hbm, attn_hbm, out_hbm,
                hn_sh, packed_v, av, rows, si, di, gs, ss_sems, as_):
    c = lax.axis_index("c")
    s = lax.axis_index("s")
    base_chunk = c * (NS * CPT0) + s * CPT1 + (1 - c) * s * (CPT0 - CPT1)
    base_chunk = pl.multiple_of(base_chunk, 32)

    brow = pl.multiple_of(base_chunk // 4, 8)
    pltpu.sync_copy(packed_hbm.at[pl.ds(brow, CPT_MAX // 4), :], packed_v)

    zero16 = jnp.zeros((16,), jnp.float32)

    def zstore(i, carry):
        rows[0][i // 8, pl.ds((i % 8) * 16, 16)] = zero16
        return carry

    lax.fori_loop(0, CHUNK * 8, zstore, 0)
    for j in range(RPT // CHUNK):
        pltpu.sync_copy(rows[0], hn_sh.at[pl.ds(s * RPT + j * CHUNK, CHUNK), :])
    plsc.subcore_barrier()

    def attn_slice(k):
        return attn_hbm.at[pl.ds((base_chunk + k) * CHUNK, CHUNK)]

    def scale(buf, k):
        pltpu.make_async_copy(attn_slice(k), av[buf], as_[buf]).wait()

        def group_body(g, carry2):
            avv = av[buf][pl.ds(g * 16, 16)]
            for j in range(16):
                e = g * 16 + j
                a = jnp.broadcast_to(avv[j], (16,))
                for db in range(D // 16):
                    sl = pl.ds(db * 16, 16)
                    rows[buf][e, sl] = rows[buf][e, sl] * a
            return carry2

        lax.fori_loop(0, CHUNK // 16, group_body, 0)

    def unpack(k, buf):
        for g in range(CHUNK // 16):
            sl = pl.ds(g * 16, 16)
            v = packed_v[k // 4, pl.ds((k % 4) * 32 + g * 16, 16)]
            si[buf][sl] = lax.shift_right_logical(v, 14)
            di[buf][sl] = lax.bitwise_and(v, jnp.int32(16383))

    def sg(k, buf):
        pltpu.async_copy(x_hbm.at[si[buf]], rows[buf], gs[buf])
        pltpu.async_copy(attn_slice(k), av[buf], as_[buf])

    def wg(buf):
        pltpu.make_async_copy(x_hbm.at[si[buf]], rows[buf], gs[buf]).wait()

    def ss(buf):
        pltpu.async_copy(rows[buf], hn_sh.at[di[buf]], ss_sems[buf], add=True)

    def ws(buf):
        pltpu.make_async_copy(rows[buf], hn_sh.at[di[buf]], ss_sems[buf]).wait()

    for k0 in range(NB - 1):
        unpack(k0, k0)
        sg(k0, k0)
    wg(0)
    scale(0, 0)
    unpack(NB - 1, NB - 1)
    sg(NB - 1, NB - 1)
    ss(0)

    def step(k, m):
        wg(m)
        scale(m, k)
        ws((m + NB - 1) % NB)
        unpack(k + NB - 1, (m + NB - 1) % NB)
        sg(k + NB - 1, (m + NB - 1) % NB)
        ss(m)

    def quad_body(kk, carry):
        for i in range(NB):
            k = NB * kk + 1 + i
            step(k, (1 + i) % NB)
        return carry

    lax.fori_loop(0, (CPT1 - NB) // NB, quad_body, 0)

    @pl.when(c == 0)
    def _():
        lax.fori_loop((CPT1 - NB) // NB, (CPT0 - NB) // NB, quad_body, 0)

    last = (CPT1 - 1) + (1 - c) * (CPT0 - CPT1)
    for i in range(NB - 1):
        k = last - (NB - 2) + i
        m = 1 + i
        wg(m)
        scale(m, k)
        ws(m - 1)
        ss(m)
    ws(NB - 1)
    plsc.subcore_barrier()

    pltpu.sync_copy(hn_sh.at[pl.ds(s * RPT, RPT), :],
                    out_hbm.at[c, pl.ds(s * RPT, RPT), :])


def _tc_body(x_ref, p_ref, w1_ref, b1_ref, w2_ref, b2_ref, o_ref):
    x = x_ref[...]
    h = p_ref[0] + p_ref[1]
    dn = (((1,), (1,)), ((), ()))
    a = lax.dot_general(x + h, w1_ref[...], dn,
                        preferred_element_type=jnp.float32) + b1_ref[...]
    b = lax.dot_general(x * h, w2_ref[...], dn,
                        preferred_element_type=jnp.float32) + b2_ref[...]
    o_ref[...] = jnp.where(a > 0, a, 0.01 * a) + jnp.where(b > 0, b, 0.01 * b)


_TC_BLK = 2000


def _bi_interaction(x, partials, W1, b1, W2, b2):
    return pl.pallas_call(
        _tc_body,
        grid=(N // _TC_BLK,),
        in_specs=[
            pl.BlockSpec((_TC_BLK, D), lambda i: (i, 0)),
            pl.BlockSpec((NC, _TC_BLK, D), lambda i: (0, i, 0)),
            pl.BlockSpec((D, D), lambda i: (0, 0)),
            pl.BlockSpec((1, D), lambda i: (0, 0)),
            pl.BlockSpec((D, D), lambda i: (0, 0)),
            pl.BlockSpec((1, D), lambda i: (0, 0)),
        ],
        out_specs=pl.BlockSpec((_TC_BLK, D), lambda i: (i, 0)),
        out_shape=jax.ShapeDtypeStruct((N, D), jnp.float32),
    )(x, partials, W1, b1.reshape(1, D), W2, b2.reshape(1, D))


def kernel(x, edge_index, edge_attn, W1, b1, W2, b2):
    src = edge_index[0]
    dst = edge_index[1]
    attn = edge_attn[:, 0]
    pad = E_ALLOC - E
    packed = src * jnp.int32(16384) + dst
    packed_p = jnp.concatenate(
        [packed, jnp.zeros((pad,), jnp.int32)]).reshape(E_ALLOC // 128, 128)
    attn_p = jnp.concatenate([attn, jnp.zeros((pad,), jnp.float32)])
    partials = _sc_message(x, packed_p, attn_p)
    return _bi_interaction(x, partials, W1, b1, W2, b2)
